# baseline (speedup 1.0000x reference)
import jax
import jax.numpy as jnp
from jax.experimental import pallas as pl
from jax.experimental.pallas import tpu as pltpu

_D_MODEL = 2048
_N_EXP = 64
_TOP_K = 2
_BLK = 2048


def _router_body(x_ref, w_ref, probs_ref, idx_ref, wts_ref):
    x = x_ref[...]
    w = w_ref[...]
    logits = jax.lax.dot_general(
        x, w, (((1,), (1,)), ((), ())), preferred_element_type=jnp.float32
    )
    m1 = jnp.max(logits, axis=1, keepdims=True)
    e = jnp.exp(logits - m1)
    s = jnp.sum(e, axis=1, keepdims=True)
    probs_ref[...] = e / s

    eidx = jax.lax.broadcasted_iota(jnp.int32, logits.shape, 1)
    i1 = jnp.min(jnp.where(logits == m1, eidx, _N_EXP), axis=1, keepdims=True)
    masked = jnp.where(eidx == i1, -jnp.inf, logits)
    m2 = jnp.max(masked, axis=1, keepdims=True)
    i2 = jnp.min(jnp.where(masked == m2, eidx, _N_EXP), axis=1, keepdims=True)

    t = jnp.exp(m2 - m1)
    w1 = 1.0 / (1.0 + t)
    w2 = t / (1.0 + t)
    idx_ref[...] = jnp.concatenate([i1, i2], axis=1)
    wts_ref[...] = jnp.concatenate([w1, w2], axis=1)


def _router(xf, W, blk=_BLK, interpret=False):
    T, D = xf.shape
    return pl.pallas_call(
        _router_body,
        grid=(T // blk,),
        in_specs=[
            pl.BlockSpec((blk, D), lambda i: (i, 0)),
            pl.BlockSpec((_N_EXP, D), lambda i: (0, 0)),
        ],
        out_specs=[
            pl.BlockSpec((blk, _N_EXP), lambda i: (i, 0)),
            pl.BlockSpec((blk, _TOP_K), lambda i: (i, 0)),
            pl.BlockSpec((blk, _TOP_K), lambda i: (i, 0)),
        ],
        out_shape=[
            jax.ShapeDtypeStruct((T, _N_EXP), jnp.float32),
            jax.ShapeDtypeStruct((T, _TOP_K), jnp.int32),
            jax.ShapeDtypeStruct((T, _TOP_K), jnp.float32),
        ],
        compiler_params=pltpu.CompilerParams(
            dimension_semantics=("parallel",),
        ),
        interpret=interpret,
    )(xf, W)


def kernel(x, W):
    B, S, D = x.shape
    T = B * S
    xf = x.reshape(T, D)
    t_main = 7 * _BLK
    pa, ia, wa = _router(xf[:t_main], W)
    pb, ib, wb = _router(xf[t_main:], W, blk=512)
    probs = jnp.concatenate([pa, pb], axis=0)
    idx = jnp.concatenate([ia, ib], axis=0)
    wts = jnp.concatenate([wa, wb], axis=0)
    return (
        probs.reshape(B, S, _N_EXP),
        idx.reshape(B, S, _TOP_K),
        wts.reshape(B, S, _TOP_K),
    )


# speedup vs baseline: 2.3593x; 2.3593x over previous
import jax
import jax.numpy as jnp
from jax.experimental import pallas as pl
from jax.experimental.pallas import tpu as pltpu

_D_MODEL = 2048
_N_EXP = 64
_TOP_K = 2
_BLK = 2048


def _router_body(x_ref, w_ref, probs_ref, idx_ref, wts_ref):
    x = x_ref[...]
    w = w_ref[...]
    logits = jax.lax.dot_general(
        x, w, (((1,), (1,)), ((), ())), preferred_element_type=jnp.float32
    )
    m1 = jnp.max(logits, axis=1, keepdims=True)
    e = jnp.exp(logits - m1)
    s = jnp.sum(e, axis=1, keepdims=True)
    probs_ref[...] = e / s

    eidx = jax.lax.broadcasted_iota(jnp.int32, logits.shape, 1)
    i1 = jnp.min(jnp.where(logits == m1, eidx, _N_EXP), axis=1, keepdims=True)
    masked = jnp.where(eidx == i1, -jnp.inf, logits)
    m2 = jnp.max(masked, axis=1, keepdims=True)
    i2 = jnp.min(jnp.where(masked == m2, eidx, _N_EXP), axis=1, keepdims=True)

    t = jnp.exp(m2 - m1)
    w1 = 1.0 / (1.0 + t)
    w2 = t / (1.0 + t)
    idx_ref[...] = jnp.concatenate([i1, i2], axis=1)
    wts_ref[...] = jnp.concatenate([w1, w2], axis=1)


def _router(xf, W, interpret=False):
    T, D = xf.shape
    return pl.pallas_call(
        _router_body,
        grid=(T // _BLK,),
        in_specs=[
            pl.BlockSpec((_BLK, D), lambda i: (i, 0)),
            pl.BlockSpec((_N_EXP, D), lambda i: (0, 0)),
        ],
        out_specs=[
            pl.BlockSpec((_BLK, _N_EXP), lambda i: (i, 0)),
            pl.BlockSpec((_BLK, _TOP_K), lambda i: (i, 0)),
            pl.BlockSpec((_BLK, _TOP_K), lambda i: (i, 0)),
        ],
        out_shape=[
            jax.ShapeDtypeStruct((T, _N_EXP), jnp.float32),
            jax.ShapeDtypeStruct((T, _TOP_K), jnp.int32),
            jax.ShapeDtypeStruct((T, _TOP_K), jnp.float32),
        ],
        compiler_params=pltpu.CompilerParams(
            dimension_semantics=("parallel",),
        ),
        interpret=interpret,
    )(xf, W)


def kernel(x, W):
    B, S, D = x.shape
    T = B * S
    probs, idx, wts = _router(x.reshape(T, D), W)
    return (
        probs.reshape(B, S, _N_EXP),
        idx.reshape(B, S, _TOP_K),
        wts.reshape(B, S, _TOP_K),
    )
